# baseline (speedup 1.0000x reference)
import numpy as np
import jax
import jax.numpy as jnp
from jax import lax
from jax.experimental import pallas as pl
from jax.experimental.pallas import tpu as pltpu

ROWS = 64
VOCAB = 1_000_000
CHUNK = 32768
NCHUNK = (VOCAB + CHUNK - 1) // CHUNK

_ROT_A = (13, 15, 26, 6)
_ROT_B = (17, 29, 16, 24)


def _np_threefry2x32(k0, k1, x0, x1):
    m = np.uint32(0xFFFFFFFF)
    ks = [np.uint32(k0), np.uint32(k1),
          np.uint32(k0) ^ np.uint32(k1) ^ np.uint32(0x1BD11BDA)]
    x0 = np.uint32(x0 + ks[0]) & m
    x1 = np.uint32(x1 + ks[1]) & m
    for i, rots in enumerate([_ROT_A, _ROT_B, _ROT_A, _ROT_B, _ROT_A]):
        for r in rots:
            x0 = np.uint32((int(x0) + int(x1)) & 0xFFFFFFFF)
            x1 = np.uint32(((int(x1) << r) | (int(x1) >> (32 - r))) & 0xFFFFFFFF)
            x1 = x0 ^ x1
        j = i + 1
        x0 = np.uint32((int(x0) + int(ks[j % 3])) & 0xFFFFFFFF)
        x1 = np.uint32((int(x1) + int(ks[(j + 1) % 3]) + j) & 0xFFFFFFFF)
    return x0, x1


_K0, _K1 = _np_threefry2x32(0, 0, 0, 1)
_K0, _K1 = np.uint32(_K0), np.uint32(_K1)
_KS = (_K0, _K1, np.uint32(_K0 ^ _K1 ^ np.uint32(0x1BD11BDA)))


def _gumbel(flat_u32):
    x0 = jnp.full_like(flat_u32, _K0)
    x1 = flat_u32 + _K1
    for i, rots in enumerate([_ROT_A, _ROT_B, _ROT_A, _ROT_B, _ROT_A]):
        for r in rots:
            x0 = x0 + x1
            x1 = (x1 << np.uint32(r)) | (x1 >> np.uint32(32 - r))
            x1 = x0 ^ x1
        j = i + 1
        x0 = x0 + _KS[j % 3]
        x1 = x1 + _KS[(j + 1) % 3] + np.uint32(j)
    bits = x0 ^ x1
    fb = (bits >> np.uint32(9)) | np.uint32(0x3F800000)
    u = lax.bitcast_convert_type(fb, jnp.float32) - jnp.float32(1.0)
    return -jnp.log(-jnp.log(u))


def _body(x_ref, out_ref, bv_ref, bi_ref):
    step = pl.program_id(0)
    base = step * CHUNK
    col = lax.broadcasted_iota(jnp.int32, (ROWS, CHUNK), 1) + base
    row = lax.broadcasted_iota(jnp.int32, (ROWS, CHUNK), 0)
    flat = (row * VOCAB + col).astype(jnp.uint32)

    y = x_ref[...] + _gumbel(flat)
    y = jnp.where(col < VOCAB, y, -jnp.inf)

    m = jnp.max(y, axis=1, keepdims=True)
    idx = jnp.min(jnp.where(y == m, col, jnp.int32(2**31 - 1)),
                  axis=1, keepdims=True)

    @pl.when(step == 0)
    def _():
        bv_ref[...] = m
        bi_ref[...] = idx

    @pl.when(step > 0)
    def _():
        better = m > bv_ref[...]
        bv_ref[...] = jnp.where(better, m, bv_ref[...])
        bi_ref[...] = jnp.where(better, idx, bi_ref[...])

    @pl.when(step == NCHUNK - 1)
    def _():
        out_ref[...] = bi_ref[...]


def kernel(x):
    out = pl.pallas_call(
        _body,
        grid=(NCHUNK,),
        in_specs=[pl.BlockSpec((ROWS, CHUNK), lambda i: (0, i))],
        out_specs=pl.BlockSpec((ROWS, 1), lambda i: (0, 0)),
        out_shape=jax.ShapeDtypeStruct((ROWS, 1), jnp.int32),
        scratch_shapes=[
            pltpu.VMEM((ROWS, 1), jnp.float32),
            pltpu.VMEM((ROWS, 1), jnp.int32),
        ],
    )(x)
    return out.reshape(ROWS)


# speedup vs baseline: 1.3774x; 1.3774x over previous
import numpy as np
import jax
import jax.numpy as jnp
from jax import lax
from jax.experimental import pallas as pl
from jax.experimental.pallas import tpu as pltpu

ROWS = 64
VOCAB = 1_000_000
CHUNK = 8192
NCHUNK = (VOCAB + CHUNK - 1) // CHUNK

_ROT_A = (13, 15, 26, 6)
_ROT_B = (17, 29, 16, 24)


def _np_threefry2x32(k0, k1, x0, x1):
    m = np.uint32(0xFFFFFFFF)
    ks = [np.uint32(k0), np.uint32(k1),
          np.uint32(k0) ^ np.uint32(k1) ^ np.uint32(0x1BD11BDA)]
    x0 = np.uint32(x0 + ks[0]) & m
    x1 = np.uint32(x1 + ks[1]) & m
    for i, rots in enumerate([_ROT_A, _ROT_B, _ROT_A, _ROT_B, _ROT_A]):
        for r in rots:
            x0 = np.uint32((int(x0) + int(x1)) & 0xFFFFFFFF)
            x1 = np.uint32(((int(x1) << r) | (int(x1) >> (32 - r))) & 0xFFFFFFFF)
            x1 = x0 ^ x1
        j = i + 1
        x0 = np.uint32((int(x0) + int(ks[j % 3])) & 0xFFFFFFFF)
        x1 = np.uint32((int(x1) + int(ks[(j + 1) % 3]) + j) & 0xFFFFFFFF)
    return x0, x1


_K0, _K1 = _np_threefry2x32(0, 0, 0, 1)
_K0, _K1 = np.uint32(_K0), np.uint32(_K1)
_KS = (_K0, _K1, np.uint32(_K0 ^ _K1 ^ np.uint32(0x1BD11BDA)))


def _gumbel(flat_u32):
    x0 = jnp.full_like(flat_u32, _K0)
    x1 = flat_u32 + _K1
    for i, rots in enumerate([_ROT_A, _ROT_B, _ROT_A, _ROT_B, _ROT_A]):
        for r in rots:
            x0 = x0 + x1
            x1 = (x1 << np.uint32(r)) | (x1 >> np.uint32(32 - r))
            x1 = x0 ^ x1
        j = i + 1
        x0 = x0 + _KS[j % 3]
        x1 = x1 + _KS[(j + 1) % 3] + np.uint32(j)
    bits = x0 ^ x1
    fb = (bits >> np.uint32(9)) | np.uint32(0x3F800000)
    u = lax.bitcast_convert_type(fb, jnp.float32) - jnp.float32(1.0)
    return -jnp.log(-jnp.log(u))


def _body(x_ref, out_ref, bv_ref, bi_ref):
    step = pl.program_id(0)
    base = step * CHUNK
    col = lax.broadcasted_iota(jnp.int32, (ROWS, CHUNK), 1) + base
    row = lax.broadcasted_iota(jnp.int32, (ROWS, CHUNK), 0)
    flat = (row * VOCAB + col).astype(jnp.uint32)

    y = x_ref[...] + _gumbel(flat)
    y = jnp.where(col < VOCAB, y, -jnp.inf)

    m = jnp.max(y, axis=1, keepdims=True)
    idx = jnp.min(jnp.where(y == m, col, jnp.int32(2**31 - 1)),
                  axis=1, keepdims=True)

    @pl.when(step == 0)
    def _():
        bv_ref[...] = m
        bi_ref[...] = idx

    @pl.when(step > 0)
    def _():
        better = m > bv_ref[...]
        bv_ref[...] = jnp.where(better, m, bv_ref[...])
        bi_ref[...] = jnp.where(better, idx, bi_ref[...])

    @pl.when(step == NCHUNK - 1)
    def _():
        out_ref[...] = bi_ref[...]


def kernel(x):
    out = pl.pallas_call(
        _body,
        grid=(NCHUNK,),
        in_specs=[pl.BlockSpec((ROWS, CHUNK), lambda i: (0, i))],
        out_specs=pl.BlockSpec((ROWS, 1), lambda i: (0, 0)),
        out_shape=jax.ShapeDtypeStruct((ROWS, 1), jnp.int32),
        scratch_shapes=[
            pltpu.VMEM((ROWS, 1), jnp.float32),
            pltpu.VMEM((ROWS, 1), jnp.int32),
        ],
    )(x)
    return out.reshape(ROWS)


# speedup vs baseline: 1.6647x; 1.2085x over previous
import numpy as np
import jax
import jax.numpy as jnp
from jax import lax
from jax.experimental import pallas as pl
from jax.experimental.pallas import tpu as pltpu

ROWS = 64
VOCAB = 1_000_000
CHUNK = 4096
NCHUNK = (VOCAB + CHUNK - 1) // CHUNK

_ROT_A = (13, 15, 26, 6)
_ROT_B = (17, 29, 16, 24)


def _np_threefry2x32(k0, k1, x0, x1):
    m = np.uint32(0xFFFFFFFF)
    ks = [np.uint32(k0), np.uint32(k1),
          np.uint32(k0) ^ np.uint32(k1) ^ np.uint32(0x1BD11BDA)]
    x0 = np.uint32(x0 + ks[0]) & m
    x1 = np.uint32(x1 + ks[1]) & m
    for i, rots in enumerate([_ROT_A, _ROT_B, _ROT_A, _ROT_B, _ROT_A]):
        for r in rots:
            x0 = np.uint32((int(x0) + int(x1)) & 0xFFFFFFFF)
            x1 = np.uint32(((int(x1) << r) | (int(x1) >> (32 - r))) & 0xFFFFFFFF)
            x1 = x0 ^ x1
        j = i + 1
        x0 = np.uint32((int(x0) + int(ks[j % 3])) & 0xFFFFFFFF)
        x1 = np.uint32((int(x1) + int(ks[(j + 1) % 3]) + j) & 0xFFFFFFFF)
    return x0, x1


_K0, _K1 = _np_threefry2x32(0, 0, 0, 1)
_K0, _K1 = np.uint32(_K0), np.uint32(_K1)
_KS = (_K0, _K1, np.uint32(_K0 ^ _K1 ^ np.uint32(0x1BD11BDA)))


def _gumbel(flat_u32):
    x0 = jnp.full_like(flat_u32, _K0)
    x1 = flat_u32 + _K1
    for i, rots in enumerate([_ROT_A, _ROT_B, _ROT_A, _ROT_B, _ROT_A]):
        for r in rots:
            x0 = x0 + x1
            x1 = (x1 << np.uint32(r)) | (x1 >> np.uint32(32 - r))
            x1 = x0 ^ x1
        j = i + 1
        x0 = x0 + _KS[j % 3]
        x1 = x1 + _KS[(j + 1) % 3] + np.uint32(j)
    bits = x0 ^ x1
    fb = (bits >> np.uint32(9)) | np.uint32(0x3F800000)
    u = lax.bitcast_convert_type(fb, jnp.float32) - jnp.float32(1.0)
    return -jnp.log(-jnp.log(u))


def _body(x_ref, out_ref, bv_ref, bi_ref):
    step = pl.program_id(0)
    base = step * CHUNK
    col = lax.broadcasted_iota(jnp.int32, (ROWS, CHUNK), 1) + base
    row = lax.broadcasted_iota(jnp.int32, (ROWS, CHUNK), 0)
    flat = (row * VOCAB + col).astype(jnp.uint32)

    y = x_ref[...] + _gumbel(flat)
    y = jnp.where(col < VOCAB, y, -jnp.inf)

    m = jnp.max(y, axis=1, keepdims=True)
    idx = jnp.min(jnp.where(y == m, col, jnp.int32(2**31 - 1)),
                  axis=1, keepdims=True)

    @pl.when(step == 0)
    def _():
        bv_ref[...] = m
        bi_ref[...] = idx

    @pl.when(step > 0)
    def _():
        better = m > bv_ref[...]
        bv_ref[...] = jnp.where(better, m, bv_ref[...])
        bi_ref[...] = jnp.where(better, idx, bi_ref[...])

    @pl.when(step == NCHUNK - 1)
    def _():
        out_ref[...] = bi_ref[...]


def kernel(x):
    out = pl.pallas_call(
        _body,
        grid=(NCHUNK,),
        in_specs=[pl.BlockSpec((ROWS, CHUNK), lambda i: (0, i))],
        out_specs=pl.BlockSpec((ROWS, 1), lambda i: (0, 0)),
        out_shape=jax.ShapeDtypeStruct((ROWS, 1), jnp.int32),
        scratch_shapes=[
            pltpu.VMEM((ROWS, 1), jnp.float32),
            pltpu.VMEM((ROWS, 1), jnp.int32),
        ],
    )(x)
    return out.reshape(ROWS)


# speedup vs baseline: 1.7897x; 1.0751x over previous
import numpy as np
import jax
import jax.numpy as jnp
from jax import lax
from jax.experimental import pallas as pl
from jax.experimental.pallas import tpu as pltpu

ROWS = 64
VOCAB = 1_000_000
CHUNK = 2048
NCHUNK = (VOCAB + CHUNK - 1) // CHUNK

_ROT_A = (13, 15, 26, 6)
_ROT_B = (17, 29, 16, 24)


def _np_threefry2x32(k0, k1, x0, x1):
    m = np.uint32(0xFFFFFFFF)
    ks = [np.uint32(k0), np.uint32(k1),
          np.uint32(k0) ^ np.uint32(k1) ^ np.uint32(0x1BD11BDA)]
    x0 = np.uint32(x0 + ks[0]) & m
    x1 = np.uint32(x1 + ks[1]) & m
    for i, rots in enumerate([_ROT_A, _ROT_B, _ROT_A, _ROT_B, _ROT_A]):
        for r in rots:
            x0 = np.uint32((int(x0) + int(x1)) & 0xFFFFFFFF)
            x1 = np.uint32(((int(x1) << r) | (int(x1) >> (32 - r))) & 0xFFFFFFFF)
            x1 = x0 ^ x1
        j = i + 1
        x0 = np.uint32((int(x0) + int(ks[j % 3])) & 0xFFFFFFFF)
        x1 = np.uint32((int(x1) + int(ks[(j + 1) % 3]) + j) & 0xFFFFFFFF)
    return x0, x1


_K0, _K1 = _np_threefry2x32(0, 0, 0, 1)
_K0, _K1 = np.uint32(_K0), np.uint32(_K1)
_KS = (_K0, _K1, np.uint32(_K0 ^ _K1 ^ np.uint32(0x1BD11BDA)))


def _gumbel(flat_u32):
    x0 = jnp.full_like(flat_u32, _K0)
    x1 = flat_u32 + _K1
    for i, rots in enumerate([_ROT_A, _ROT_B, _ROT_A, _ROT_B, _ROT_A]):
        for r in rots:
            x0 = x0 + x1
            x1 = (x1 << np.uint32(r)) | (x1 >> np.uint32(32 - r))
            x1 = x0 ^ x1
        j = i + 1
        x0 = x0 + _KS[j % 3]
        x1 = x1 + _KS[(j + 1) % 3] + np.uint32(j)
    bits = x0 ^ x1
    fb = (bits >> np.uint32(9)) | np.uint32(0x3F800000)
    u = lax.bitcast_convert_type(fb, jnp.float32) - jnp.float32(1.0)
    return -jnp.log(-jnp.log(u))


def _body(x_ref, out_ref, bv_ref, bi_ref):
    step = pl.program_id(0)
    base = step * CHUNK
    col = lax.broadcasted_iota(jnp.int32, (ROWS, CHUNK), 1) + base
    row = lax.broadcasted_iota(jnp.int32, (ROWS, CHUNK), 0)
    flat = (row * VOCAB + col).astype(jnp.uint32)

    y = x_ref[...] + _gumbel(flat)
    y = jnp.where(col < VOCAB, y, -jnp.inf)

    m = jnp.max(y, axis=1, keepdims=True)
    idx = jnp.min(jnp.where(y == m, col, jnp.int32(2**31 - 1)),
                  axis=1, keepdims=True)

    @pl.when(step == 0)
    def _():
        bv_ref[...] = m
        bi_ref[...] = idx

    @pl.when(step > 0)
    def _():
        better = m > bv_ref[...]
        bv_ref[...] = jnp.where(better, m, bv_ref[...])
        bi_ref[...] = jnp.where(better, idx, bi_ref[...])

    @pl.when(step == NCHUNK - 1)
    def _():
        out_ref[...] = bi_ref[...]


def kernel(x):
    out = pl.pallas_call(
        _body,
        grid=(NCHUNK,),
        in_specs=[pl.BlockSpec((ROWS, CHUNK), lambda i: (0, i))],
        out_specs=pl.BlockSpec((ROWS, 1), lambda i: (0, 0)),
        out_shape=jax.ShapeDtypeStruct((ROWS, 1), jnp.int32),
        scratch_shapes=[
            pltpu.VMEM((ROWS, 1), jnp.float32),
            pltpu.VMEM((ROWS, 1), jnp.int32),
        ],
    )(x)
    return out.reshape(ROWS)
